# baseline (speedup 1.0000x reference)
import functools
import math

import jax
import jax.numpy as jnp
from jax import lax
from jax.experimental import pallas as pl
from jax.experimental.pallas import tpu as pltpu
from jax.experimental.pallas import tpu_sc as plsc

ROWS = 16
COLS = 2048
L = 16
UNROLL = 4
NCHUNK = COLS // (L * UNROLL)
INV_N = 1.0 / (ROWS * COLS)

_mesh = plsc.VectorSubcoreMesh(core_axis_name="c", subcore_axis_name="s",
                               num_cores=1)


def _lane_allreduce(vec):
    lane = lax.iota(jnp.int32, L)
    for off in (1, 2, 4, 8):
        vec = vec + vec.at[lane ^ off].get(
            mode=lax.GatherScatterMode.PROMISE_IN_BOUNDS)
    return vec


@functools.partial(
    pl.kernel,
    mesh=_mesh,
    out_type=jax.ShapeDtypeStruct((L,), jnp.float32),
    scratch_types=[
        pltpu.VMEM((COLS,), jnp.float32),
        pltpu.VMEM((COLS,), jnp.float32),
        pltpu.VMEM((L,), jnp.float32),
        pltpu.VMEM((ROWS * L,), jnp.float32),
        pltpu.VMEM_SHARED((ROWS * L,), jnp.float32),
        pltpu.SemaphoreType.DMA,
        pltpu.SemaphoreType.DMA,
    ],
)
def _corr_mse_kernel(pred_hbm, label_hbm, out_hbm,
                     pred_v, label_v, part_v, fin_v, shared,
                     sem_p, sem_l):
    s = lax.axis_index("s")

    if True:
        cp_p = pltpu.async_copy(pred_hbm.at[s], pred_v, sem_p)
        cp_l = pltpu.async_copy(label_hbm.at[s], label_v, sem_l)
        cp_p.wait()
        cp_l.wait()

        zero = jnp.zeros((L,), jnp.float32)

        @plsc.parallel_loop(0, COLS // L, unroll=UNROLL,
                            carry=(zero, zero, zero, zero))
        def loop_carry(j, carry):
            sse, spos, sneg, nones = carry
            base = j * L
            p = pred_v[pl.ds(base, L)]
            lab = label_v[pl.ds(base, L)]
            nlab = 1.0 - lab
            d = p - lab
            t = jnp.exp(p * (nlab - lab))
            sse = sse + d * d
            spos = spos + lab * t
            sneg = sneg + nlab * t
            nones = nones + lab
            return sse, spos, sneg, nones

        sse, spos, sneg, nones = loop_carry

        sse_t = _lane_allreduce(sse)
        s_pos = _lane_allreduce(spos)
        s_neg = _lane_allreduce(sneg)
        n_one = _lane_allreduce(nones)
        n_zero = float(COLS) - n_one

        loss_both = s_pos * s_neg / jnp.maximum(n_one * n_zero, 1.0)
        loss_all_zero = s_neg * math.exp(-1.0) / jnp.maximum(n_zero, 1.0)
        loss_all_one = s_pos / jnp.maximum(n_one, 1.0)
        row_loss = jnp.where(
            n_one == 0.0, loss_all_zero,
            jnp.where(n_zero == 0.0, loss_all_one, loss_both))

        part_v[...] = row_loss + sse_t * INV_N
        pltpu.sync_copy(part_v, shared.at[pl.ds(s * L, L)])

    plsc.subcore_barrier()

    @pl.when(s == 0)
    def _combine():
        pltpu.sync_copy(shared, fin_v)
        acc = fin_v[pl.ds(0, L)]
        for i in range(1, ROWS):
            acc = acc + fin_v[pl.ds(i * L, L)]
        part_v[...] = acc
        pltpu.sync_copy(part_v, out_hbm)


def kernel(pred, label):
    out = _corr_mse_kernel(pred, label)
    return out[0]


# speedup vs baseline: 1.0005x; 1.0005x over previous
import functools
import math

import jax
import jax.numpy as jnp
from jax import lax
from jax.experimental import pallas as pl
from jax.experimental.pallas import tpu as pltpu
from jax.experimental.pallas import tpu_sc as plsc

ROWS = 16
COLS = 2048
L = 16
UNROLL = 8
HALF = COLS // 2
INV_N = 1.0 / (ROWS * COLS)

_mesh = plsc.VectorSubcoreMesh(core_axis_name="c", subcore_axis_name="s",
                               num_cores=1)


def _lane_allreduce(vec):
    lane = lax.iota(jnp.int32, L)
    for off in (1, 2, 4, 8):
        vec = vec + vec.at[lane ^ off].get(
            mode=lax.GatherScatterMode.PROMISE_IN_BOUNDS)
    return vec


@functools.partial(
    pl.kernel,
    mesh=_mesh,
    out_type=jax.ShapeDtypeStruct((L,), jnp.float32),
    scratch_types=[
        pltpu.VMEM((COLS,), jnp.float32),
        pltpu.VMEM((COLS,), jnp.float32),
        pltpu.VMEM((L,), jnp.float32),
        pltpu.VMEM((ROWS * L,), jnp.float32),
        pltpu.VMEM_SHARED((ROWS * L,), jnp.float32),
        pltpu.SemaphoreType.DMA,
        pltpu.SemaphoreType.DMA,
        pltpu.SemaphoreType.DMA,
        pltpu.SemaphoreType.DMA,
    ],
)
def _corr_mse_kernel(pred_hbm, label_hbm, out_hbm,
                     pred_v, label_v, part_v, fin_v, shared,
                     sem_p0, sem_l0, sem_p1, sem_l1):
    s = lax.axis_index("s")

    if True:
        cp_p0 = pltpu.async_copy(pred_hbm.at[s, pl.ds(0, HALF)],
                                 pred_v.at[pl.ds(0, HALF)], sem_p0)
        cp_l0 = pltpu.async_copy(label_hbm.at[s, pl.ds(0, HALF)],
                                 label_v.at[pl.ds(0, HALF)], sem_l0)
        cp_p1 = pltpu.async_copy(pred_hbm.at[s, pl.ds(HALF, HALF)],
                                 pred_v.at[pl.ds(HALF, HALF)], sem_p1)
        cp_l1 = pltpu.async_copy(label_hbm.at[s, pl.ds(HALF, HALF)],
                                 label_v.at[pl.ds(HALF, HALF)], sem_l1)

        zero = jnp.zeros((L,), jnp.float32)

        def half_loop(lo, carry_in):
            @plsc.parallel_loop(lo, lo + HALF // L, unroll=UNROLL,
                                carry=carry_in)
            def loop_carry(j, carry):
                sse, spos, sneg, nones = carry
                base = j * L
                p = pred_v[pl.ds(base, L)]
                lab = label_v[pl.ds(base, L)]
                nlab = 1.0 - lab
                d = p - lab
                t = jnp.exp(p * (nlab - lab))
                sse = sse + d * d
                spos = spos + lab * t
                sneg = sneg + nlab * t
                nones = nones + lab
                return sse, spos, sneg, nones
            return loop_carry

        cp_p0.wait()
        cp_l0.wait()
        carry = half_loop(0, (zero, zero, zero, zero))
        cp_p1.wait()
        cp_l1.wait()
        sse, spos, sneg, nones = half_loop(HALF // L, carry)

        sse_t = _lane_allreduce(sse)
        s_pos = _lane_allreduce(spos)
        s_neg = _lane_allreduce(sneg)
        n_one = _lane_allreduce(nones)
        n_zero = float(COLS) - n_one

        loss_both = s_pos * s_neg / jnp.maximum(n_one * n_zero, 1.0)
        loss_all_zero = s_neg * math.exp(-1.0) / jnp.maximum(n_zero, 1.0)
        loss_all_one = s_pos / jnp.maximum(n_one, 1.0)
        row_loss = jnp.where(
            n_one == 0.0, loss_all_zero,
            jnp.where(n_zero == 0.0, loss_all_one, loss_both))

        part_v[...] = row_loss + sse_t * INV_N
        pltpu.sync_copy(part_v, shared.at[pl.ds(s * L, L)])

    plsc.subcore_barrier()

    @pl.when(s == 0)
    def _combine():
        pltpu.sync_copy(shared, fin_v)
        acc = fin_v[pl.ds(0, L)]
        for i in range(1, ROWS):
            acc = acc + fin_v[pl.ds(i * L, L)]
        part_v[...] = acc
        pltpu.sync_copy(part_v, out_hbm)


def kernel(pred, label):
    out = _corr_mse_kernel(pred, label)
    return out[0]
